# Initial kernel scaffold; baseline (speedup 1.0000x reference)
import jax
import jax.numpy as jnp
from jax.experimental import pallas as pl


def kernel(x, noise):
    raise NotImplementedError("write your pallas kernel here")


# speedup vs baseline: 6.7642x; 6.7642x over previous
import jax
import jax.numpy as jnp
from jax.experimental import pallas as pl

_MASK_RATIO = 0.75


def _mask_kernel(noise_ref, mask_ref, *, k):
    bits = jax.lax.bitcast_convert_type(noise_ref[...], jnp.int32)
    d, l = bits.shape

    def body(_, carry):
        lo, hi = carry
        mid = lo + (hi - lo) // 2
        cnt = jnp.sum((bits <= mid).astype(jnp.int32), axis=1, keepdims=True)
        ge = cnt >= k
        return jnp.where(ge, lo, mid + 1), jnp.where(ge, mid, hi)

    lo = jnp.zeros((d, 1), jnp.int32)
    hi = jnp.full((d, 1), 1 << 30, jnp.int32)
    lo, hi = jax.lax.fori_loop(0, 30, body, (lo, hi))
    thresh = lo

    lt = bits < thresh
    eq = bits == thresh
    cnt_lt = jnp.sum(lt.astype(jnp.int32), axis=1, keepdims=True)
    ties_to_keep = (k - cnt_lt).astype(jnp.float32)

    row = jax.lax.broadcasted_iota(jnp.int32, (l, l), 0)
    col = jax.lax.broadcasted_iota(jnp.int32, (l, l), 1)
    tri = (row < col).astype(jnp.float32)
    prefix_eq = jax.lax.dot(eq.astype(jnp.float32), tri,
                            preferred_element_type=jnp.float32)
    keep = lt | (eq & (prefix_eq < ties_to_keep))
    mask_ref[...] = keep.astype(jnp.float32)


def _mul_kernel(x_ref, mask_ref, out_ref):
    out_ref[...] = x_ref[...] * mask_ref[...]


def kernel(x, noise):
    d, c, h, w = x.shape
    l = h * w
    k = int(l * (1 - _MASK_RATIO))
    x3 = x.reshape(d, c, l)

    mask = pl.pallas_call(
        lambda nr, mr: _mask_kernel(nr, mr, k=k),
        out_shape=jax.ShapeDtypeStruct((d, l), jnp.float32),
    )(noise)

    out3 = pl.pallas_call(
        _mul_kernel,
        grid=(d,),
        in_specs=[
            pl.BlockSpec((1, c, l), lambda i: (i, 0, 0)),
            pl.BlockSpec((1, 1, l), lambda i: (i, 0, 0)),
        ],
        out_specs=pl.BlockSpec((1, c, l), lambda i: (i, 0, 0)),
        out_shape=jax.ShapeDtypeStruct((d, c, l), x.dtype),
    )(x3, mask.reshape(d, 1, l))

    return out3.reshape(d, c, h, w)


# speedup vs baseline: 9.3877x; 1.3879x over previous
import jax
import jax.numpy as jnp
from jax.experimental import pallas as pl
from jax.experimental.pallas import tpu as pltpu

_MASK_RATIO = 0.75


def _mask_kernel(noise_ref, mask_ref, *, k):
    bits = jax.lax.bitcast_convert_type(noise_ref[...], jnp.int32)
    d, l = bits.shape

    def body(_, carry):
        lo, hi = carry
        mid = lo + (hi - lo) // 2
        cnt = jnp.sum((bits <= mid).astype(jnp.int32), axis=1, keepdims=True)
        ge = cnt >= k
        return jnp.where(ge, lo, mid + 1), jnp.where(ge, mid, hi)

    lo = jnp.zeros((d, 1), jnp.int32)
    hi = jnp.full((d, 1), 1 << 30, jnp.int32)
    lo, hi = jax.lax.fori_loop(0, 30, body, (lo, hi))
    thresh = lo

    lt = bits < thresh
    eq = bits == thresh
    cnt_lt = jnp.sum(lt.astype(jnp.int32), axis=1, keepdims=True)
    ties_to_keep = (k - cnt_lt).astype(jnp.float32)

    row = jax.lax.broadcasted_iota(jnp.int32, (l, l), 0)
    col = jax.lax.broadcasted_iota(jnp.int32, (l, l), 1)
    tri = (row < col).astype(jnp.float32)
    prefix_eq = jax.lax.dot(eq.astype(jnp.float32), tri,
                            preferred_element_type=jnp.float32)
    keep = lt | (eq & (prefix_eq < ties_to_keep))
    mask_ref[...] = keep.astype(jnp.float32)


def _mul_kernel(x_ref, mask_ref, out_ref):
    out_ref[...] = x_ref[...] * mask_ref[...]


def kernel(x, noise):
    d, c, h, w = x.shape
    l = h * w
    k = int(l * (1 - _MASK_RATIO))
    x3 = x.reshape(d, c, l)

    mask = pl.pallas_call(
        lambda nr, mr: _mask_kernel(nr, mr, k=k),
        out_shape=jax.ShapeDtypeStruct((d, l), jnp.float32),
    )(noise)

    bd = 8
    out3 = pl.pallas_call(
        _mul_kernel,
        grid=(d // bd,),
        in_specs=[
            pl.BlockSpec((bd, c, l), lambda i: (i, 0, 0)),
            pl.BlockSpec((bd, 1, l), lambda i: (i, 0, 0)),
        ],
        out_specs=pl.BlockSpec((bd, c, l), lambda i: (i, 0, 0)),
        out_shape=jax.ShapeDtypeStruct((d, c, l), x.dtype),
        compiler_params=pltpu.CompilerParams(
            dimension_semantics=("parallel",),
        ),
    )(x3, mask.reshape(d, 1, l))

    return out3.reshape(d, c, h, w)


# speedup vs baseline: 9.4744x; 1.0092x over previous
import jax
import jax.numpy as jnp
from jax.experimental import pallas as pl
from jax.experimental.pallas import tpu as pltpu

_MASK_RATIO = 0.75


def _mask_kernel(noise_ref, mask_ref, *, k):
    bits = jax.lax.bitcast_convert_type(noise_ref[...], jnp.int32)
    d, l = bits.shape

    def body(_, carry):
        lo, hi = carry
        mid = lo + (hi - lo) // 2
        cnt = jnp.sum((bits <= mid).astype(jnp.int32), axis=1, keepdims=True)
        ge = cnt >= k
        return jnp.where(ge, lo, mid + 1), jnp.where(ge, mid, hi)

    lo = jnp.zeros((d, 1), jnp.int32)
    hi = jnp.full((d, 1), 1 << 30, jnp.int32)
    lo, hi = jax.lax.fori_loop(0, 30, body, (lo, hi))
    thresh = lo

    lt = bits < thresh
    eq = bits == thresh
    cnt_lt = jnp.sum(lt.astype(jnp.int32), axis=1, keepdims=True)
    ties_to_keep = (k - cnt_lt).astype(jnp.float32)

    row = jax.lax.broadcasted_iota(jnp.int32, (l, l), 0)
    col = jax.lax.broadcasted_iota(jnp.int32, (l, l), 1)
    tri = (row < col).astype(jnp.float32)
    prefix_eq = jax.lax.dot(eq.astype(jnp.float32), tri,
                            preferred_element_type=jnp.float32)
    keep = lt | (eq & (prefix_eq < ties_to_keep))
    mask_ref[...] = keep.astype(jnp.float32)


def _mul_kernel(x_ref, mask_ref, out_ref):
    out_ref[...] = x_ref[...] * mask_ref[...]


def kernel(x, noise):
    d, c, h, w = x.shape
    l = h * w
    k = int(l * (1 - _MASK_RATIO))
    x3 = x.reshape(d, c, l)

    mask = pl.pallas_call(
        lambda nr, mr: _mask_kernel(nr, mr, k=k),
        out_shape=jax.ShapeDtypeStruct((d, l), jnp.float32),
    )(noise)

    bd = 16
    out3 = pl.pallas_call(
        _mul_kernel,
        grid=(d // bd,),
        in_specs=[
            pl.BlockSpec((bd, c, l), lambda i: (i, 0, 0)),
            pl.BlockSpec((bd, 1, l), lambda i: (i, 0, 0)),
        ],
        out_specs=pl.BlockSpec((bd, c, l), lambda i: (i, 0, 0)),
        out_shape=jax.ShapeDtypeStruct((d, c, l), x.dtype),
        compiler_params=pltpu.CompilerParams(
            dimension_semantics=("parallel",),
        ),
    )(x3, mask.reshape(d, 1, l))

    return out3.reshape(d, c, h, w)
